# Initial kernel scaffold; baseline (speedup 1.0000x reference)
import jax
import jax.numpy as jnp
from jax.experimental import pallas as pl


def kernel(x, durations, max_length):
    raise NotImplementedError("write your pallas kernel here")


# speedup vs baseline: 101.8824x; 101.8824x over previous
import functools

import jax
import jax.numpy as jnp
from jax import lax
from jax.experimental import pallas as pl
from jax.experimental.pallas import tpu as pltpu

_M = 2048
_FT = 1024


def _lr_body(ml_ref, dur_ref, x_ref, out_ref, mask_ref):
    f = pl.program_id(1)
    S = dur_ref.shape[-1]
    d = jnp.round(dur_ref[0])
    rows = lax.broadcasted_iota(jnp.int32, (S, S), 0)
    cols = lax.broadcasted_iota(jnp.int32, (S, S), 1)
    tri = (rows <= cols).astype(jnp.float32)
    cs = jnp.dot(d, tri, preferred_element_type=jnp.float32)
    cs = jnp.minimum(cs, ml_ref[0, 0])
    prev = jnp.concatenate(
        [jnp.zeros((1, 1), jnp.float32), cs[:, : S - 1]], axis=1)
    base = (f * _FT).astype(jnp.float32)
    tf = base + lax.broadcasted_iota(jnp.int32, (_FT, S), 0).astype(jnp.float32)
    onehot = ((tf >= prev) & (tf < cs)).astype(jnp.float32)
    out_ref[0] = jnp.dot(onehot, x_ref[0], preferred_element_type=jnp.float32)
    total = jnp.max(cs)
    tm = base + lax.broadcasted_iota(jnp.int32, (1, _FT), 1).astype(jnp.float32)
    mask_ref[0, 0] = (tm >= total).astype(jnp.int32)


def kernel(x, durations, max_length):
    B, S, C = x.shape
    dur3 = durations.reshape(B, 1, S)
    grid = (B, _M // _FT)
    ml = jnp.asarray(max_length, jnp.float32).reshape(1, 1)
    expanded, mask_i = pl.pallas_call(
        _lr_body,
        grid=grid,
        in_specs=[
            pl.BlockSpec((1, 1), lambda b, f: (0, 0),
                         memory_space=pltpu.MemorySpace.SMEM),
            pl.BlockSpec((1, 1, S), lambda b, f: (b, 0, 0)),
            pl.BlockSpec((1, S, C), lambda b, f: (b, 0, 0)),
        ],
        out_specs=[
            pl.BlockSpec((1, _FT, C), lambda b, f: (b, f, 0)),
            pl.BlockSpec((1, 1, 1, _FT), lambda b, f: (b, f, 0, 0)),
        ],
        out_shape=[
            jax.ShapeDtypeStruct((B, _M, C), jnp.float32),
            jax.ShapeDtypeStruct((B, _M // _FT, 1, _FT), jnp.int32),
        ],
        compiler_params=pltpu.CompilerParams(
            dimension_semantics=("parallel", "parallel")),
    )(ml, dur3, x)
    mel_masks = mask_i.reshape(B, _M).astype(bool)
    return expanded, mel_masks
